# baseline (speedup 1.0000x reference)
import functools

import jax
import jax.numpy as jnp
from jax import lax
from jax.experimental import pallas as pl
from jax.experimental.pallas import tpu as pltpu
from jax.experimental.pallas import tpu_sc as plsc

K = 16
NPG = 100
NPAD = 112
NGRP = NPAD // 16
BIG = 1 << 30
INV_DEG = 1.0 / 17.0
NWORKERS = 32


def _mm1_kernel(x_ref, w_ref, o_ref):
    o_ref[...] = jnp.dot(x_ref[...], w_ref[...],
                         preferred_element_type=jnp.float32)


def _mm2_kernel(g_ref, w_ref, b_ref, o_ref):
    o_ref[...] = (jnp.dot(g_ref[...], w_ref[...],
                          preferred_element_type=jnp.float32) * INV_DEG
                  + b_ref[...])


def _sc_body(xs_hbm, ys_hbm, xsb_hbm, ysb_hbm, xw1_hbm, b1b_hbm, g2_hbm,
             xs_v, ys_v, xsb_v, ysb_v, xw1_v, h_v, g2_v, idxT_v, b1b_v):
    B = xs_hbm.shape[0]
    wid = lax.axis_index("s") * 2 + lax.axis_index("c")

    pltpu.sync_copy(b1b_hbm, b1b_v)
    lane = lax.iota(jnp.int32, 16)

    for gi0 in range(NGRP):
        plsc.store_scatter(
            g2_v, [lane + 16 * gi0, jnp.full((16,), K, jnp.int32)],
            jnp.zeros((16,), jnp.float32))

    def _merge(a, b):
        ak, av = a
        bk, bv = b
        bk2 = lax.rev(bk, (0,))
        bv2 = lax.rev(bv, (0,))
        ta = ak <= bk2
        ck = jnp.where(ta, ak, bk2)
        cv = jnp.where(ta, av, bv2)
        return plsc.sort_key_val(ck, cv)

    def _process_graph(g):
        pltpu.sync_copy(xs_hbm.at[g], xs_v)
        pltpu.sync_copy(ys_hbm.at[g], ys_v)
        pltpu.sync_copy(xsb_hbm.at[g], xsb_v)
        pltpu.sync_copy(ysb_hbm.at[g], ysb_v)
        pltpu.sync_copy(xw1_hbm.at[g], xw1_v.at[pl.ds(0, NPG)])
        for n in range(K):
            idxT_v[n, pl.ds(NPG - 4, 16)] = jnp.zeros((16,), jnp.int32)

        @plsc.parallel_loop(0, NPG, 1, unroll=4)
        def _node(i):
            ii = jnp.full((16,), i, jnp.int32)
            xi = xsb_v[i]
            yi = ysb_v[i]
            groups = []
            for j in range(NGRP):
                xg = xs_v[pl.ds(16 * j, 16)]
                yg = ys_v[pl.ds(16 * j, 16)]
                dx = xg - xi
                dy = yg - yi
                d = dx * dx + dy * dy
                jv = lane + (16 * j)
                key = d * 128 + jv
                if 16 * (j + 1) > NPG:
                    key = jnp.where(jv >= NPG, BIG, key)
                key = jnp.where(jv == i, BIG, key)
                groups.append(plsc.sort_key_val(key, jv))
            m01 = _merge(groups[0], groups[1])
            m23 = _merge(groups[2], groups[3])
            m45 = _merge(groups[4], groups[5])
            m0123 = _merge(m01, m23)
            ak, av = _merge(m45, groups[6])
            bk = lax.rev(ak, (0,))
            bv = lax.rev(av, (0,))
            ta = m0123[0] <= bk
            va = jnp.where(ta, m0123[1], bv)
            plsc.store_scatter(idxT_v, [lane, ii], va)

        def _agg(gi, src_v, dst_v, relu_bias):
            base = gi * 16
            nodes = lane + base
            accs = [plsc.load_gather(src_v, [nodes, jnp.full((16,), f, jnp.int32)])
                    for f in range(K)]
            for n in range(K):
                idx_n = idxT_v[n, pl.ds(base, 16)]
                for f in range(K):
                    accs[f] = accs[f] + plsc.load_gather(
                        src_v, [idx_n, jnp.full((16,), f, jnp.int32)])
            for f in range(K):
                v = accs[f]
                if relu_bias:
                    v = jnp.maximum(v * INV_DEG + b1b_v[f], 0.0)
                plsc.store_scatter(
                    dst_v, [nodes, jnp.full((16,), f, jnp.int32)], v)
            return 0

        @plsc.parallel_loop(0, NGRP, 1)
        def _agg1(gi):
            _agg(gi, xw1_v, h_v, True)

        @plsc.parallel_loop(0, NGRP, 1)
        def _agg2(gi):
            _agg(gi, h_v, g2_v, False)

        pltpu.sync_copy(g2_v.at[pl.ds(0, NPG)], g2_hbm.at[g])

    def _step(t, _):
        g = wid + NWORKERS * t

        @pl.when(g < B)
        def _():
            _process_graph(g)

        return 0

    lax.fori_loop(0, (B + NWORKERS - 1) // NWORKERS, _step, 0)


@jax.jit
def kernel(coo, x, W1, b1, W2, b2):
    N = x.shape[0]
    B = N // NPG
    d_in = x.shape[1]
    d_hid = W1.shape[1]
    d_out = W2.shape[1]

    blk = 2000
    xw1 = pl.pallas_call(
        _mm1_kernel,
        grid=(N // blk,),
        in_specs=[
            pl.BlockSpec((blk, d_in), lambda i: (i, 0)),
            pl.BlockSpec((d_in, d_hid), lambda i: (0, 0)),
        ],
        out_specs=pl.BlockSpec((blk, d_hid), lambda i: (i, 0)),
        out_shape=jax.ShapeDtypeStruct((N, d_hid), jnp.float32),
    )(x, W1)

    xs = jnp.zeros((B, NPAD), jnp.int32).at[:, :NPG].set(
        coo[:, 0].reshape(B, NPG))
    ys = jnp.zeros((B, NPAD), jnp.int32).at[:, :NPG].set(
        coo[:, 1].reshape(B, NPG))
    xsb = jnp.broadcast_to(xs[:, :, None], (B, NPAD, 16))
    ysb = jnp.broadcast_to(ys[:, :, None], (B, NPAD, 16))
    xw1r = jnp.zeros((B, NPG, d_hid + 1), jnp.float32).at[:, :, :d_hid].set(
        xw1.reshape(B, NPG, d_hid))
    b1b = jnp.broadcast_to(b1[:, None], (d_hid, 16))

    mesh = plsc.VectorSubcoreMesh(core_axis_name="c", subcore_axis_name="s",
                                  num_cores=2, num_subcores=16)
    g2 = pl.kernel(
        _sc_body,
        out_type=jax.ShapeDtypeStruct((B, NPG, d_hid + 1), jnp.float32),
        mesh=mesh,
        compiler_params=pltpu.CompilerParams(needs_layout_passes=False),
        scratch_types=[
            pltpu.VMEM((NPAD,), jnp.int32),
            pltpu.VMEM((NPAD,), jnp.int32),
            pltpu.VMEM((NPAD, 16), jnp.int32),
            pltpu.VMEM((NPAD, 16), jnp.int32),
            pltpu.VMEM((NPAD, d_hid + 1), jnp.float32),
            pltpu.VMEM((NPAD, d_hid + 1), jnp.float32),
            pltpu.VMEM((NPAD, d_hid + 1), jnp.float32),
            pltpu.VMEM((K, NPAD + 1), jnp.int32),
            pltpu.VMEM((d_hid, 16), jnp.float32),
        ],
    )(xs, ys, xsb, ysb, xw1r, b1b)

    w2p = jnp.zeros((d_hid + 1, d_out), jnp.float32).at[:d_hid].set(W2)
    out = pl.pallas_call(
        _mm2_kernel,
        grid=(N // blk,),
        in_specs=[
            pl.BlockSpec((blk, d_hid + 1), lambda i: (i, 0)),
            pl.BlockSpec((d_hid + 1, d_out), lambda i: (0, 0)),
            pl.BlockSpec((1, d_out), lambda i: (0, 0)),
        ],
        out_specs=pl.BlockSpec((blk, d_out), lambda i: (i, 0)),
        out_shape=jax.ShapeDtypeStruct((N, d_out), jnp.float32),
    )(g2.reshape(N, d_hid + 1), w2p, b2[None])
    return out


# speedup vs baseline: 2.1573x; 2.1573x over previous
import functools

import jax
import jax.numpy as jnp
from jax import lax
from jax.experimental import pallas as pl
from jax.experimental.pallas import tpu as pltpu
from jax.experimental.pallas import tpu_sc as plsc

K = 16
NPG = 100
NPAD = 112
NGRP = NPAD // 16
BIG = 1 << 30
INV_DEG = 1.0 / 17.0
NWORKERS = 32
FW = 17


def _mm1_kernel(x_ref, w_ref, o_ref):
    o_ref[...] = jnp.dot(x_ref[...], w_ref[...],
                         preferred_element_type=jnp.float32)


def _mm2_kernel(g_ref, w_ref, b_ref, o_ref):
    o_ref[...] = (jnp.dot(g_ref[...], w_ref[...],
                          preferred_element_type=jnp.float32) * INV_DEG
                  + b_ref[...])


def _sc_body(xs_hbm, ys_hbm, xw1_hbm, b1b_hbm, g2_hbm,
             xs_v, ys_v, xw1_v, h_v, g2_v, idxT_v, b1b_v):
    B = xs_hbm.shape[0]
    wid = lax.axis_index("s") * 2 + lax.axis_index("c")

    pltpu.sync_copy(b1b_hbm, b1b_v)
    lane = lax.iota(jnp.int32, 16)

    for gi0 in range(NGRP):
        plsc.store_scatter(
            g2_v, [(lane + 16 * gi0) * FW + K], jnp.zeros((16,), jnp.float32))

    def _merge(a, b):
        ak, av = a
        bk, bv = b
        bk2 = lax.rev(bk, (0,))
        bv2 = lax.rev(bv, (0,))
        ta = ak <= bk2
        ck = jnp.where(ta, ak, bk2)
        cv = jnp.where(ta, av, bv2)
        return plsc.sort_key_val(ck, cv)

    def _process_graph(g):
        pltpu.sync_copy(xs_hbm.at[g], xs_v)
        pltpu.sync_copy(ys_hbm.at[g], ys_v)
        pltpu.sync_copy(xw1_hbm.at[g], xw1_v)
        for n in range(K):
            idxT_v[n, pl.ds(NPG - 4, 16)] = jnp.zeros((16,), jnp.int32)

        def _node(i, _):
            ii = jnp.full((16,), i, jnp.int32)
            xi = plsc.load_gather(xs_v, [ii])
            yi = plsc.load_gather(ys_v, [ii])
            groups = []
            for j in range(NGRP):
                xg = xs_v[pl.ds(16 * j, 16)]
                yg = ys_v[pl.ds(16 * j, 16)]
                dx = xg - xi
                dy = yg - yi
                d = dx * dx + dy * dy
                jv = lane + (16 * j)
                key = d * 128 + jv
                if 16 * (j + 1) > NPG:
                    key = jnp.where(jv >= NPG, BIG, key)
                key = jnp.where(jv == i, BIG, key)
                groups.append(plsc.sort_key_val(key, jv))
            m01 = _merge(groups[0], groups[1])
            m23 = _merge(groups[2], groups[3])
            m45 = _merge(groups[4], groups[5])
            m0123 = _merge(m01, m23)
            m456 = _merge(m45, groups[6])
            _, va = _merge(m0123, m456)
            plsc.store_scatter(idxT_v, [lane, ii], va)
            return 0

        lax.fori_loop(0, NPG, _node, 0)

        def _agg(gi, src_v, dst_v, relu_bias):
            base = gi * 16
            nflat = (lane + base) * FW
            accs = [plsc.load_gather(src_v, [nflat + f]) for f in range(K)]
            for n in range(K):
                ib = idxT_v[n, pl.ds(base, 16)] * FW
                for f in range(K):
                    accs[f] = accs[f] + plsc.load_gather(src_v, [ib + f])
            for f in range(K):
                v = accs[f]
                if relu_bias:
                    v = jnp.maximum(v * INV_DEG + b1b_v[f], 0.0)
                plsc.store_scatter(dst_v, [nflat + f], v)
            return 0

        lax.fori_loop(0, NGRP, lambda gi, c: _agg(gi, xw1_v, h_v, True), 0)
        lax.fori_loop(0, NGRP, lambda gi, c: _agg(gi, h_v, g2_v, False), 0)

        pltpu.sync_copy(g2_v, g2_hbm.at[g])

    def _step(t, _):
        g = wid + NWORKERS * t

        @pl.when(g < B)
        def _():
            _process_graph(g)

        return 0

    lax.fori_loop(0, (B + NWORKERS - 1) // NWORKERS, _step, 0)


@jax.jit
def kernel(coo, x, W1, b1, W2, b2):
    N = x.shape[0]
    B = N // NPG
    d_in = x.shape[1]
    d_hid = W1.shape[1]
    d_out = W2.shape[1]

    blk = 2000
    xw1 = pl.pallas_call(
        _mm1_kernel,
        grid=(N // blk,),
        in_specs=[
            pl.BlockSpec((blk, d_in), lambda i: (i, 0)),
            pl.BlockSpec((d_in, d_hid), lambda i: (0, 0)),
        ],
        out_specs=pl.BlockSpec((blk, d_hid), lambda i: (i, 0)),
        out_shape=jax.ShapeDtypeStruct((N, d_hid), jnp.float32),
    )(x, W1)

    xs = jnp.zeros((B, NPAD), jnp.int32).at[:, :NPG].set(
        coo[:, 0].reshape(B, NPG))
    ys = jnp.zeros((B, NPAD), jnp.int32).at[:, :NPG].set(
        coo[:, 1].reshape(B, NPG))
    xw1r = jnp.zeros((B, NPAD, FW), jnp.float32).at[:, :NPG, :d_hid].set(
        xw1.reshape(B, NPG, d_hid)).reshape(B, NPAD * FW)
    b1b = jnp.broadcast_to(b1[:, None], (d_hid, 16))

    mesh = plsc.VectorSubcoreMesh(core_axis_name="c", subcore_axis_name="s",
                                  num_cores=2, num_subcores=16)
    g2 = pl.kernel(
        _sc_body,
        out_type=jax.ShapeDtypeStruct((B, NPAD * FW), jnp.float32),
        mesh=mesh,
        compiler_params=pltpu.CompilerParams(needs_layout_passes=False),
        scratch_types=[
            pltpu.VMEM((NPAD,), jnp.int32),
            pltpu.VMEM((NPAD,), jnp.int32),
            pltpu.VMEM((NPAD * FW,), jnp.float32),
            pltpu.VMEM((NPAD * FW,), jnp.float32),
            pltpu.VMEM((NPAD * FW,), jnp.float32),
            pltpu.VMEM((K, NPAD + 1), jnp.int32),
            pltpu.VMEM((d_hid, 16), jnp.float32),
        ],
    )(xs, ys, xw1r, b1b)

    w2p = jnp.zeros((FW, d_out), jnp.float32).at[:d_hid].set(W2)
    out = pl.pallas_call(
        _mm2_kernel,
        grid=(N // blk,),
        in_specs=[
            pl.BlockSpec((blk, FW), lambda i: (i, 0)),
            pl.BlockSpec((FW, d_out), lambda i: (0, 0)),
            pl.BlockSpec((1, d_out), lambda i: (0, 0)),
        ],
        out_specs=pl.BlockSpec((blk, d_out), lambda i: (i, 0)),
        out_shape=jax.ShapeDtypeStruct((N, d_out), jnp.float32),
    )(g2.reshape(B, NPAD, FW)[:, :NPG].reshape(N, FW), w2p, b2[None])
    return out
